# baseline (speedup 1.0000x reference)
import functools

import jax
import jax.numpy as jnp
from jax import lax
from jax.experimental import pallas as pl
from jax.experimental.pallas import tpu as pltpu
from jax.experimental.pallas import tpu_sc as plsc

N = 10000
NE = 320000
NODE = 128
H = 64
DT = 0.05
TAU_MIN = 0.01

NC = 2
NS = 16
NW = NC * NS
CH = 128
NCHUNK = NE // CH
F32 = jnp.float32
_DBLK = 200
_NDB = N // _DBLK

def _mesh():
    return plsc.VectorSubcoreMesh(core_axis_name="c", subcore_axis_name="s",
                                  num_cores=NC, num_subcores=NS)


def _preproj_body(v_ref, w_ref, a_ref, b_ref):
    ab = jnp.dot(v_ref[...], w_ref[...], preferred_element_type=F32)
    a_ref[...] = ab[:, :NODE]
    b_ref[...] = ab[:, NODE:]


def _preproj(V, Wsr):
    blk = 2000
    return pl.pallas_call(
        _preproj_body,
        grid=(N // blk,),
        in_specs=[
            pl.BlockSpec((blk, NODE), lambda i: (i, 0)),
            pl.BlockSpec((NODE, 2 * NODE), lambda i: (0, 0)),
        ],
        out_specs=[
            pl.BlockSpec((blk, NODE), lambda i: (i, 0)),
            pl.BlockSpec((blk, NODE), lambda i: (i, 0)),
        ],
        out_shape=[
            jax.ShapeDtypeStruct((N, NODE), F32),
            jax.ShapeDtypeStruct((N, NODE), F32),
        ],
    )(V, Wsr)


def _gather_body(nchunk, a_hbm, b_hbm, src_hbm, dst_hbm, sab_hbm,
                 idxs0, idxd0, ra0, idxs1, idxd1, ra1,
                 sl0, sg0, sw0, sl1, sg1, sw1):
    c = lax.axis_index("c")
    s = lax.axis_index("s")
    wid = s * NC + c
    n = (nchunk - wid + NW - 1) // NW
    slots = ((idxs0, idxd0, ra0, sl0, sg0, sw0),
             (idxs1, idxd1, ra1, sl1, sg1, sw1))

    def issue_loads(k, slot):
        idxs, idxd, _, sL, _, _ = slot
        base = (wid + k * NW) * CH
        pltpu.async_copy(src_hbm.at[pl.ds(base, CH)], idxs, sL)
        pltpu.async_copy(dst_hbm.at[pl.ds(base, CH)], idxd, sL)

    issue_loads(0, slots[0])
    issue_loads(1, slots[1])

    def grp(g, carry):
        for b in range(2):
            idxs, idxd, ra, sL, sG, sW = slots[b]
            k = g * 2 + b

            @pl.when(k < n)
            def _do():
                base = (wid + k * NW) * CH
                hbm_out = sab_hbm.at[pl.ds(base, CH)]
                pltpu.make_async_copy(src_hbm.at[pl.ds(base, CH)],
                                      idxs, sL).wait()
                pltpu.make_async_copy(dst_hbm.at[pl.ds(base, CH)],
                                      idxd, sL).wait()

                @pl.when(k >= 2)
                def _drain_wb():
                    pltpu.make_async_copy(ra, hbm_out, sW).wait()

                pltpu.async_copy(a_hbm.at[idxs], ra, sG).wait()
                pltpu.async_copy(b_hbm.at[idxd], ra, sG, add=True).wait()
                pltpu.async_copy(ra, hbm_out, sW)

                @pl.when(k + 2 < n)
                def _prefetch():
                    issue_loads(k + 2, slots[b])
        return carry

    lax.fori_loop(0, (n + 1) // 2, grp, 0)
    for b in range(2):
        _, _, ra, _, _, sW = slots[b]
        pltpu.make_async_copy(ra, sab_hbm.at[pl.ds(0, CH)], sW).wait()


def _sc_gather(A, B, src, dst):
    ne = src.shape[0]
    k = functools.partial(
        pl.kernel,
        out_type=jax.ShapeDtypeStruct((ne, NODE), F32),
        mesh=_mesh(),
        scratch_types=[
            pltpu.VMEM((CH,), jnp.int32),
            pltpu.VMEM((CH,), jnp.int32),
            pltpu.VMEM((CH, NODE), F32),
            pltpu.VMEM((CH,), jnp.int32),
            pltpu.VMEM((CH,), jnp.int32),
            pltpu.VMEM((CH, NODE), F32),
            pltpu.SemaphoreType.DMA,
            pltpu.SemaphoreType.DMA,
            pltpu.SemaphoreType.DMA,
            pltpu.SemaphoreType.DMA,
            pltpu.SemaphoreType.DMA,
            pltpu.SemaphoreType.DMA,
        ],
    )(functools.partial(_gather_body, ne // CH))
    return k(A, B, src, dst)


def _edge_mlp_body(sab_ref, e_ref,
                   w1e_ref, b1_ref, w2_ref, b2_ref,
                   out_ref, sc0_ref, sc1_ref):
    pre = (sab_ref[...]
           + jnp.dot(e_ref[...], w1e_ref[...], preferred_element_type=F32)
           + b1_ref[...])
    h = pre * jax.nn.sigmoid(pre)
    emb = (jnp.dot(h, w2_ref[...], preferred_element_type=F32)
           + b2_ref[...])
    out_ref[...] = emb
    n = emb.shape[0]
    one = jnp.ones((n, 1), F32)
    zer = jnp.zeros((n, NODE - H - 1), F32)
    sc0_ref[...] = jnp.concatenate([emb[:, :H], one, zer], axis=1)
    sc1_ref[...] = jnp.concatenate([emb[:, H:], one, zer], axis=1)


def _edge_mlp(SAB, E, eoff, W1e, b1, W2, b2):
    ne = SAB.shape[0]
    blk = 1280
    row = lambda i: (i, 0)
    erow = lambda i: (i + eoff // blk, 0)
    full = lambda i: (0, 0)
    return pl.pallas_call(
        _edge_mlp_body,
        grid=(ne // blk,),
        in_specs=[
            pl.BlockSpec((blk, NODE), row),
            pl.BlockSpec((blk, NODE), erow),
            pl.BlockSpec((NODE, NODE), full),
            pl.BlockSpec((1, NODE), full),
            pl.BlockSpec((NODE, NODE), full),
            pl.BlockSpec((1, NODE), full),
        ],
        out_specs=[
            pl.BlockSpec((blk, NODE), row),
            pl.BlockSpec((blk, NODE), row),
            pl.BlockSpec((blk, NODE), row),
        ],
        out_shape=[
            jax.ShapeDtypeStruct((ne, NODE), F32),
            jax.ShapeDtypeStruct((ne, NODE), F32),
            jax.ShapeDtypeStruct((ne, NODE), F32),
        ],
    )(SAB, E, W1e, b1, W2, b2)


def _remap_body(src_ref, dst_ref, i00_ref, i01_ref, i10_ref, i11_ref):
    nh = N // 2
    sidx = src_ref[...]
    didx = dst_ref[...]
    i00_ref[...] = jnp.where(sidx < nh, sidx, nh)
    i01_ref[...] = jnp.where(sidx >= nh, sidx - nh, nh)
    i10_ref[...] = jnp.where(didx < nh, didx, nh)
    i11_ref[...] = jnp.where(didx >= nh, didx - nh, nh)


def _remap(src, dst):
    nr = NE // CH
    shp = jax.ShapeDtypeStruct((nr, CH), jnp.int32)
    return pl.pallas_call(
        _remap_body,
        out_shape=[shp, shp, shp, shp],
    )(src.reshape(nr, CH), dst.reshape(nr, CH))


_NHALF = N // 2
_ACC_R = _NHALF + _DBLK
_NZB = _ACC_R // _DBLK
_NDB2 = _NHALF // _DBLK


def _scatter_body(nchunk, sc0_hbm, sc1_hbm, i00_hbm, i01_hbm, i10_hbm,
                  i11_hbm, acc_hbm, idx0_v, ev0_v, idx1_v, ev1_v,
                  idx2_v, ev2_v, sl0, sl1, sl2, sa0, sa1, sa2,
                  z_v, sh_acc):
    c = lax.axis_index("c")
    s = lax.axis_index("s")
    n = (nchunk - s + NS - 1) // NS
    slots = ((idx0_v, ev0_v, sl0, sa0), (idx1_v, ev1_v, sl1, sa1),
             (idx2_v, ev2_v, sl2, sa2))

    zero16 = jnp.zeros((16,), F32)

    def fill_z(r, carry):
        for j in range(8):
            z_v[r, pl.ds(j * 16, 16)] = zero16
        return carry

    lax.fori_loop(0, _DBLK, fill_z, 0)

    for p, (idxA_hbm, idxB_hbm) in enumerate(((i00_hbm, i10_hbm),
                                              (i01_hbm, i11_hbm))):
        for j in range((_NZB + NS - 1) // NS):
            b = s + NS * j

            @pl.when(b < _NZB)
            def _zero():
                pltpu.sync_copy(z_v, sh_acc.at[pl.ds(b * _DBLK, _DBLK)])
        plsc.subcore_barrier()

        def issue_loads(k, slot):
            idx_v, ev_v, sL, _ = slot
            base = (s + k * NS) * CH

            @pl.when(c == 0)
            def _load0():
                pltpu.async_copy(idxA_hbm.at[pl.ds(base, CH)], idx_v, sL)
                pltpu.async_copy(sc0_hbm.at[pl.ds(base, CH)], ev_v, sL)

            @pl.when(c == 1)
            def _load1():
                pltpu.async_copy(idxB_hbm.at[pl.ds(base, CH)], idx_v, sL)
                pltpu.async_copy(sc1_hbm.at[pl.ds(base, CH)], ev_v, sL)

        issue_loads(0, slots[0])

        def grp(g, carry):
            for b in range(3):
                idx_v, ev_v, sL, sA = slots[b]
                idx2, ev2, _, sA2 = slots[(b + 1) % 3]
                k = g * 3 + b

                @pl.when(k < n)
                def _do():
                    base = (s + k * NS) * CH
                    pltpu.make_async_copy(idxA_hbm.at[pl.ds(base, CH)],
                                          idx_v, sL).wait()
                    pltpu.make_async_copy(sc0_hbm.at[pl.ds(base, CH)],
                                          ev_v, sL).wait()
                    pltpu.async_copy(ev_v, sh_acc.at[idx_v], sA, add=True)

                    @pl.when(k >= 2)
                    def _drain_add():
                        pltpu.make_async_copy(ev2, sh_acc.at[idx2],
                                              sA2).wait()

                    @pl.when(k + 1 < n)
                    def _prefetch():
                        issue_loads(k + 1, slots[(b + 1) % 3])
            return carry

        lax.fori_loop(0, (n + 2) // 3, grp, 0)
        for d in (2, 1):
            bb = (n - d) % 3

            @pl.when(bb == 0)
            def _d0():
                pltpu.make_async_copy(ev0_v, sh_acc.at[idx0_v], sa0).wait()

            @pl.when(bb == 1)
            def _d1():
                pltpu.make_async_copy(ev1_v, sh_acc.at[idx1_v], sa1).wait()

            @pl.when(bb == 2)
            def _d2():
                pltpu.make_async_copy(ev2_v, sh_acc.at[idx2_v], sa2).wait()
        plsc.subcore_barrier()

        for j in range((_NDB2 + NS - 1) // NS):
            b = s + NS * j

            @pl.when(b < _NDB2)
            def _dump():
                r = b * _DBLK
                pltpu.sync_copy(sh_acc.at[pl.ds(r, _DBLK)],
                                acc_hbm.at[c, pl.ds(p * _NHALF + r, _DBLK)])
        plsc.subcore_barrier()


def _sc_scatter(sc0, sc1, i00, i01, i10, i11):
    k = functools.partial(
        pl.kernel,
        out_type=jax.ShapeDtypeStruct((NC, N, NODE), F32),
        mesh=_mesh(),
        scratch_types=[
            pltpu.VMEM((CH,), jnp.int32),
            pltpu.VMEM((CH, NODE), F32),
            pltpu.VMEM((CH,), jnp.int32),
            pltpu.VMEM((CH, NODE), F32),
            pltpu.VMEM((CH,), jnp.int32),
            pltpu.VMEM((CH, NODE), F32),
            pltpu.SemaphoreType.DMA,
            pltpu.SemaphoreType.DMA,
            pltpu.SemaphoreType.DMA,
            pltpu.SemaphoreType.DMA,
            pltpu.SemaphoreType.DMA,
            pltpu.SemaphoreType.DMA,
            pltpu.VMEM((_DBLK, NODE), F32),
            pltpu.VMEM_SHARED((_ACC_R, NODE), F32),
        ],
    )(functools.partial(_scatter_body, sc0.shape[0] // CH))
    return k(sc0, sc1, i00, i01, i10, i11)


def _node_body(v_ref, acc_ref,
               wep_ref, bep_ref, wt_ref, bt_ref, wg_ref, bg_ref,
               wf1_ref, bf1_ref, wf2_ref, bf2_ref, gam_ref, bet_ref,
               out_ref):
    x = v_ref[...]
    nparts = acc_ref.shape[0]
    a0 = acc_ref[0].astype(F32)
    a1 = acc_ref[1].astype(F32)
    for t in range(2, nparts, 2):
        a0 = a0 + acc_ref[t].astype(F32)
        a1 = a1 + acc_ref[t + 1].astype(F32)
    m0 = a0[:, :H] / jnp.maximum(a0[:, H:H + 1], 1.0)
    m1 = a1[:, :H] / jnp.maximum(a1[:, H:H + 1], 1.0)
    em = jnp.concatenate([m0, m1], axis=-1)
    u = jnp.dot(em, wep_ref[...], preferred_element_type=F32) + bep_ref[...]
    xu = jnp.concatenate([x, u], axis=-1)
    tau = jnp.maximum(
        jax.nn.softplus(jnp.dot(xu, wt_ref[...], preferred_element_type=F32)
                        + bt_ref[...]), TAU_MIN)
    gate = jax.nn.sigmoid(jnp.dot(xu, wg_ref[...], preferred_element_type=F32)
                          + bg_ref[...])
    h1 = jnp.dot(xu, wf1_ref[...], preferred_element_type=F32) + bf1_ref[...]
    h1 = h1 * jax.nn.sigmoid(h1)
    f = jnp.tanh(jnp.dot(h1, wf2_ref[...], preferred_element_type=F32)
                 + bf2_ref[...]) * gate
    xn = x + (1.0 / tau) * (-x + f) * DT
    mu = jnp.mean(xn, axis=-1, keepdims=True)
    var = jnp.mean((xn - mu) ** 2, axis=-1, keepdims=True)
    out_ref[...] = ((xn - mu) / jnp.sqrt(var + 1e-5) * gam_ref[...]
                    + bet_ref[...])


def _node_update(V, acc, Wep, bep, Wt, bt, Wg, bg,
                 Wf1, bf1, Wf2, bf2, gamma, beta):
    blk = 2000
    row = lambda i: (i, 0)
    part = lambda i: (0, i, 0)
    full = lambda i: (0, 0)
    d2 = 2 * NODE
    return pl.pallas_call(
        _node_body,
        grid=(N // blk,),
        in_specs=[
            pl.BlockSpec((blk, NODE), row),
            pl.BlockSpec((acc.shape[0], blk, NODE), part),
            pl.BlockSpec((NODE, NODE), full),
            pl.BlockSpec((1, NODE), full),
            pl.BlockSpec((d2, NODE), full),
            pl.BlockSpec((1, NODE), full),
            pl.BlockSpec((d2, NODE), full),
            pl.BlockSpec((1, NODE), full),
            pl.BlockSpec((d2, d2), full),
            pl.BlockSpec((1, d2), full),
            pl.BlockSpec((d2, NODE), full),
            pl.BlockSpec((1, NODE), full),
            pl.BlockSpec((1, NODE), full),
            pl.BlockSpec((1, NODE), full),
        ],
        out_specs=pl.BlockSpec((blk, NODE), row),
        out_shape=jax.ShapeDtypeStruct((N, NODE), F32),
    )(V, acc, Wep, bep, Wt, bt, Wg, bg,
      Wf1, bf1, Wf2, bf2, gamma, beta)


def kernel(V, E, edges, We1, be1, We2, be2, Wep, bep,
           Wf1, bf1, Wf2, bf2, Wt, bt, Wg, bg, gamma, beta):
    V2 = V[0]
    E2 = E[0]
    src = edges[0, :, 0]
    dst = edges[0, :, 1]
    Wsr = jnp.concatenate([We1[:NODE], We1[NODE:2 * NODE]], axis=1)
    W1e = We1[2 * NODE:]

    A, B = _preproj(V2, Wsr)
    i00, i01, i10, i11 = _remap(src, dst)

    K = 2
    neh = NE // K
    nrh = neh // CH
    embs, accs = [], []
    for t in range(K):
        sl = slice(t * neh, (t + 1) * neh)
        rsl = slice(t * nrh, (t + 1) * nrh)
        SAB = _sc_gather(A, B, src[sl], dst[sl])
        emb_t, sc0, sc1 = _edge_mlp(SAB, E2, t * neh, W1e,
                                    be1.reshape(1, -1), We2,
                                    be2.reshape(1, -1))
        acc_t = _sc_scatter(sc0, sc1,
                            i00[rsl].reshape(neh), i01[rsl].reshape(neh),
                            i10[rsl].reshape(neh), i11[rsl].reshape(neh))
        embs.append(emb_t)
        accs.append(acc_t)
    emb = jnp.concatenate(embs, axis=0)
    acc = jnp.concatenate(accs, axis=0)
    node_emb = _node_update(
        V2, acc, Wep, bep.reshape(1, -1), Wt, bt.reshape(1, -1),
        Wg, bg.reshape(1, -1), Wf1, bf1.reshape(1, -1),
        Wf2, bf2.reshape(1, -1), gamma.reshape(1, -1), beta.reshape(1, -1))
    return node_emb[None], emb[None]


# speedup vs baseline: 1.0498x; 1.0498x over previous
import functools

import jax
import jax.numpy as jnp
from jax import lax
from jax.experimental import pallas as pl
from jax.experimental.pallas import tpu as pltpu
from jax.experimental.pallas import tpu_sc as plsc

N = 10000
NE = 320000
NODE = 128
H = 64
DT = 0.05
TAU_MIN = 0.01

NC = 2
NS = 16
NW = NC * NS
CH = 128
NCHUNK = NE // CH
F32 = jnp.float32
_DBLK = 200
_NDB = N // _DBLK

def _mesh():
    return plsc.VectorSubcoreMesh(core_axis_name="c", subcore_axis_name="s",
                                  num_cores=NC, num_subcores=NS)


def _preproj_body(v_ref, w_ref, a_ref, b_ref):
    ab = jnp.dot(v_ref[...], w_ref[...], preferred_element_type=F32)
    a_ref[...] = ab[:, :NODE]
    b_ref[...] = ab[:, NODE:]


def _preproj(V, Wsr):
    blk = 2000
    return pl.pallas_call(
        _preproj_body,
        grid=(N // blk,),
        in_specs=[
            pl.BlockSpec((blk, NODE), lambda i: (i, 0)),
            pl.BlockSpec((NODE, 2 * NODE), lambda i: (0, 0)),
        ],
        out_specs=[
            pl.BlockSpec((blk, NODE), lambda i: (i, 0)),
            pl.BlockSpec((blk, NODE), lambda i: (i, 0)),
        ],
        out_shape=[
            jax.ShapeDtypeStruct((N, NODE), F32),
            jax.ShapeDtypeStruct((N, NODE), F32),
        ],
    )(V, Wsr)


def _gather_body(nchunk, a_hbm, b_hbm, src_hbm, dst_hbm, sab_hbm,
                 idxs0, idxd0, ra0, idxs1, idxd1, ra1,
                 sl0, sg0, sw0, sl1, sg1, sw1):
    c = lax.axis_index("c")
    s = lax.axis_index("s")
    wid = s * NC + c
    n = (nchunk - wid + NW - 1) // NW
    slots = ((idxs0, idxd0, ra0, sl0, sg0, sw0),
             (idxs1, idxd1, ra1, sl1, sg1, sw1))

    def issue_loads(k, slot):
        idxs, idxd, _, sL, _, _ = slot
        base = (wid + k * NW) * CH
        pltpu.async_copy(src_hbm.at[pl.ds(base, CH)], idxs, sL)
        pltpu.async_copy(dst_hbm.at[pl.ds(base, CH)], idxd, sL)

    issue_loads(0, slots[0])
    issue_loads(1, slots[1])

    def grp(g, carry):
        for b in range(2):
            idxs, idxd, ra, sL, sG, sW = slots[b]
            k = g * 2 + b

            @pl.when(k < n)
            def _do():
                base = (wid + k * NW) * CH
                hbm_out = sab_hbm.at[pl.ds(base, CH)]
                pltpu.make_async_copy(src_hbm.at[pl.ds(base, CH)],
                                      idxs, sL).wait()
                pltpu.make_async_copy(dst_hbm.at[pl.ds(base, CH)],
                                      idxd, sL).wait()

                @pl.when(k >= 2)
                def _drain_wb():
                    pltpu.make_async_copy(ra, hbm_out, sW).wait()

                pltpu.async_copy(a_hbm.at[idxs], ra, sG).wait()
                pltpu.async_copy(b_hbm.at[idxd], ra, sG, add=True).wait()
                pltpu.async_copy(ra, hbm_out, sW)

                @pl.when(k + 2 < n)
                def _prefetch():
                    issue_loads(k + 2, slots[b])
        return carry

    lax.fori_loop(0, (n + 1) // 2, grp, 0)
    for b in range(2):
        _, _, ra, _, _, sW = slots[b]
        pltpu.make_async_copy(ra, sab_hbm.at[pl.ds(0, CH)], sW).wait()


def _sc_gather(A, B, src, dst):
    ne = src.shape[0]
    k = functools.partial(
        pl.kernel,
        out_type=jax.ShapeDtypeStruct((ne, NODE), F32),
        mesh=_mesh(),
        scratch_types=[
            pltpu.VMEM((CH,), jnp.int32),
            pltpu.VMEM((CH,), jnp.int32),
            pltpu.VMEM((CH, NODE), F32),
            pltpu.VMEM((CH,), jnp.int32),
            pltpu.VMEM((CH,), jnp.int32),
            pltpu.VMEM((CH, NODE), F32),
            pltpu.SemaphoreType.DMA,
            pltpu.SemaphoreType.DMA,
            pltpu.SemaphoreType.DMA,
            pltpu.SemaphoreType.DMA,
            pltpu.SemaphoreType.DMA,
            pltpu.SemaphoreType.DMA,
        ],
    )(functools.partial(_gather_body, ne // CH))
    return k(A, B, src, dst)


def _edge_mlp_body(sab_ref, e_ref,
                   w1e_ref, b1_ref, w2_ref, b2_ref,
                   out_ref, sc0_ref, sc1_ref):
    pre = (sab_ref[...]
           + jnp.dot(e_ref[...], w1e_ref[...], preferred_element_type=F32)
           + b1_ref[...])
    h = pre * jax.nn.sigmoid(pre)
    emb = (jnp.dot(h, w2_ref[...], preferred_element_type=F32)
           + b2_ref[...])
    out_ref[...] = emb
    n = emb.shape[0]
    one = jnp.ones((n, 1), F32)
    zer = jnp.zeros((n, NODE - H - 1), F32)
    sc0_ref[...] = jnp.concatenate([emb[:, :H], one, zer], axis=1)
    sc1_ref[...] = jnp.concatenate([emb[:, H:], one, zer], axis=1)


def _edge_mlp(SAB, E, eoff, W1e, b1, W2, b2):
    ne = SAB.shape[0]
    blk = 1280
    row = lambda i: (i, 0)
    erow = lambda i: (i + eoff // blk, 0)
    full = lambda i: (0, 0)
    return pl.pallas_call(
        _edge_mlp_body,
        grid=(ne // blk,),
        in_specs=[
            pl.BlockSpec((blk, NODE), row),
            pl.BlockSpec((blk, NODE), erow),
            pl.BlockSpec((NODE, NODE), full),
            pl.BlockSpec((1, NODE), full),
            pl.BlockSpec((NODE, NODE), full),
            pl.BlockSpec((1, NODE), full),
        ],
        out_specs=[
            pl.BlockSpec((blk, NODE), row),
            pl.BlockSpec((blk, NODE), row),
            pl.BlockSpec((blk, NODE), row),
        ],
        out_shape=[
            jax.ShapeDtypeStruct((ne, NODE), F32),
            jax.ShapeDtypeStruct((ne, NODE), F32),
            jax.ShapeDtypeStruct((ne, NODE), F32),
        ],
    )(SAB, E, W1e, b1, W2, b2)


def _remap_body(src_ref, dst_ref, i00_ref, i01_ref, i10_ref, i11_ref):
    nh = N // 2
    sidx = src_ref[...]
    didx = dst_ref[...]
    i00_ref[...] = jnp.where(sidx < nh, sidx, nh)
    i01_ref[...] = jnp.where(sidx >= nh, sidx - nh, nh)
    i10_ref[...] = jnp.where(didx < nh, didx, nh)
    i11_ref[...] = jnp.where(didx >= nh, didx - nh, nh)


def _remap(src, dst):
    nr = NE // CH
    shp = jax.ShapeDtypeStruct((nr, CH), jnp.int32)
    return pl.pallas_call(
        _remap_body,
        out_shape=[shp, shp, shp, shp],
    )(src.reshape(nr, CH), dst.reshape(nr, CH))


_NHALF = N // 2
_ACC_R = _NHALF + _DBLK
_NZB = _ACC_R // _DBLK
_NDB2 = _NHALF // _DBLK


def _scatter_body(nchunk, sc0_hbm, sc1_hbm, i00_hbm, i01_hbm, i10_hbm,
                  i11_hbm, acc_hbm, idx0_v, ev0_v, idx1_v, ev1_v, sl0, sl1,
                  z_v, sh_acc):
    c = lax.axis_index("c")
    s = lax.axis_index("s")
    n = (nchunk - s + NS - 1) // NS
    slots = ((idx0_v, ev0_v, sl0), (idx1_v, ev1_v, sl1))

    zero16 = jnp.zeros((16,), F32)

    def fill_z(r, carry):
        for j in range(8):
            z_v[r, pl.ds(j * 16, 16)] = zero16
        return carry

    lax.fori_loop(0, _DBLK, fill_z, 0)

    for p, (idxA_hbm, idxB_hbm) in enumerate(((i00_hbm, i10_hbm),
                                              (i01_hbm, i11_hbm))):
        for j in range((_NZB + NS - 1) // NS):
            b = s + NS * j

            @pl.when(b < _NZB)
            def _zero():
                pltpu.sync_copy(z_v, sh_acc.at[pl.ds(b * _DBLK, _DBLK)])
        plsc.subcore_barrier()

        def issue_loads(k, slot):
            idx_v, ev_v, sL = slot
            base = (s + k * NS) * CH

            @pl.when(c == 0)
            def _load0():
                pltpu.async_copy(idxA_hbm.at[pl.ds(base, CH)], idx_v, sL)
                pltpu.async_copy(sc0_hbm.at[pl.ds(base, CH)], ev_v, sL)

            @pl.when(c == 1)
            def _load1():
                pltpu.async_copy(idxB_hbm.at[pl.ds(base, CH)], idx_v, sL)
                pltpu.async_copy(sc1_hbm.at[pl.ds(base, CH)], ev_v, sL)

        issue_loads(0, slots[0])
        issue_loads(1, slots[1])

        def grp(g, carry):
            for b in range(2):
                idx_v, ev_v, sL = slots[b]
                k = g * 2 + b

                @pl.when(k < n)
                def _do():
                    base = (s + k * NS) * CH
                    pltpu.make_async_copy(idxA_hbm.at[pl.ds(base, CH)],
                                          idx_v, sL).wait()
                    pltpu.make_async_copy(sc0_hbm.at[pl.ds(base, CH)],
                                          ev_v, sL).wait()
                    pltpu.sync_copy(ev_v, sh_acc.at[idx_v], add=True)

                    @pl.when(k + 2 < n)
                    def _prefetch():
                        issue_loads(k + 2, slots[b])
            return carry

        lax.fori_loop(0, (n + 1) // 2, grp, 0)
        plsc.subcore_barrier()

        for j in range((_NDB2 + NS - 1) // NS):
            b = s + NS * j

            @pl.when(b < _NDB2)
            def _dump():
                r = b * _DBLK
                pltpu.sync_copy(sh_acc.at[pl.ds(r, _DBLK)],
                                acc_hbm.at[c, pl.ds(p * _NHALF + r, _DBLK)])
        plsc.subcore_barrier()


def _sc_scatter(sc0, sc1, i00, i01, i10, i11):
    k = functools.partial(
        pl.kernel,
        out_type=jax.ShapeDtypeStruct((NC, N, NODE), F32),
        mesh=_mesh(),
        scratch_types=[
            pltpu.VMEM((CH,), jnp.int32),
            pltpu.VMEM((CH, NODE), F32),
            pltpu.VMEM((CH,), jnp.int32),
            pltpu.VMEM((CH, NODE), F32),
            pltpu.SemaphoreType.DMA,
            pltpu.SemaphoreType.DMA,
            pltpu.VMEM((_DBLK, NODE), F32),
            pltpu.VMEM_SHARED((_ACC_R, NODE), F32),
        ],
    )(functools.partial(_scatter_body, sc0.shape[0] // CH))
    return k(sc0, sc1, i00, i01, i10, i11)


def _node_body(v_ref, acc_ref,
               wep_ref, bep_ref, wt_ref, bt_ref, wg_ref, bg_ref,
               wf1_ref, bf1_ref, wf2_ref, bf2_ref, gam_ref, bet_ref,
               out_ref):
    x = v_ref[...]
    nparts = acc_ref.shape[0]
    a0 = acc_ref[0].astype(F32)
    a1 = acc_ref[1].astype(F32)
    for t in range(2, nparts, 2):
        a0 = a0 + acc_ref[t].astype(F32)
        a1 = a1 + acc_ref[t + 1].astype(F32)
    m0 = a0[:, :H] / jnp.maximum(a0[:, H:H + 1], 1.0)
    m1 = a1[:, :H] / jnp.maximum(a1[:, H:H + 1], 1.0)
    em = jnp.concatenate([m0, m1], axis=-1)
    u = jnp.dot(em, wep_ref[...], preferred_element_type=F32) + bep_ref[...]
    xu = jnp.concatenate([x, u], axis=-1)
    tau = jnp.maximum(
        jax.nn.softplus(jnp.dot(xu, wt_ref[...], preferred_element_type=F32)
                        + bt_ref[...]), TAU_MIN)
    gate = jax.nn.sigmoid(jnp.dot(xu, wg_ref[...], preferred_element_type=F32)
                          + bg_ref[...])
    h1 = jnp.dot(xu, wf1_ref[...], preferred_element_type=F32) + bf1_ref[...]
    h1 = h1 * jax.nn.sigmoid(h1)
    f = jnp.tanh(jnp.dot(h1, wf2_ref[...], preferred_element_type=F32)
                 + bf2_ref[...]) * gate
    xn = x + (1.0 / tau) * (-x + f) * DT
    mu = jnp.mean(xn, axis=-1, keepdims=True)
    var = jnp.mean((xn - mu) ** 2, axis=-1, keepdims=True)
    out_ref[...] = ((xn - mu) / jnp.sqrt(var + 1e-5) * gam_ref[...]
                    + bet_ref[...])


def _node_update(V, acc, Wep, bep, Wt, bt, Wg, bg,
                 Wf1, bf1, Wf2, bf2, gamma, beta):
    blk = 2000
    row = lambda i: (i, 0)
    part = lambda i: (0, i, 0)
    full = lambda i: (0, 0)
    d2 = 2 * NODE
    return pl.pallas_call(
        _node_body,
        grid=(N // blk,),
        in_specs=[
            pl.BlockSpec((blk, NODE), row),
            pl.BlockSpec((acc.shape[0], blk, NODE), part),
            pl.BlockSpec((NODE, NODE), full),
            pl.BlockSpec((1, NODE), full),
            pl.BlockSpec((d2, NODE), full),
            pl.BlockSpec((1, NODE), full),
            pl.BlockSpec((d2, NODE), full),
            pl.BlockSpec((1, NODE), full),
            pl.BlockSpec((d2, d2), full),
            pl.BlockSpec((1, d2), full),
            pl.BlockSpec((d2, NODE), full),
            pl.BlockSpec((1, NODE), full),
            pl.BlockSpec((1, NODE), full),
            pl.BlockSpec((1, NODE), full),
        ],
        out_specs=pl.BlockSpec((blk, NODE), row),
        out_shape=jax.ShapeDtypeStruct((N, NODE), F32),
    )(V, acc, Wep, bep, Wt, bt, Wg, bg,
      Wf1, bf1, Wf2, bf2, gamma, beta)


def kernel(V, E, edges, We1, be1, We2, be2, Wep, bep,
           Wf1, bf1, Wf2, bf2, Wt, bt, Wg, bg, gamma, beta):
    V2 = V[0]
    E2 = E[0]
    src = edges[0, :, 0]
    dst = edges[0, :, 1]
    Wsr = jnp.concatenate([We1[:NODE], We1[NODE:2 * NODE]], axis=1)
    W1e = We1[2 * NODE:]

    A, B = _preproj(V2, Wsr)
    i00, i01, i10, i11 = _remap(src, dst)

    K = 2
    neh = NE // K
    nrh = neh // CH
    embs, accs = [], []
    for t in range(K):
        sl = slice(t * neh, (t + 1) * neh)
        rsl = slice(t * nrh, (t + 1) * nrh)
        SAB = _sc_gather(A, B, src[sl], dst[sl])
        emb_t, sc0, sc1 = _edge_mlp(SAB, E2, t * neh, W1e,
                                    be1.reshape(1, -1), We2,
                                    be2.reshape(1, -1))
        acc_t = _sc_scatter(sc0, sc1,
                            i00[rsl].reshape(neh), i01[rsl].reshape(neh),
                            i10[rsl].reshape(neh), i11[rsl].reshape(neh))
        embs.append(emb_t)
        accs.append(acc_t)
    emb = jnp.concatenate(embs, axis=0)
    acc = jnp.concatenate(accs, axis=0)
    node_emb = _node_update(
        V2, acc, Wep, bep.reshape(1, -1), Wt, bt.reshape(1, -1),
        Wg, bg.reshape(1, -1), Wf1, bf1.reshape(1, -1),
        Wf2, bf2.reshape(1, -1), gamma.reshape(1, -1), beta.reshape(1, -1))
    return node_emb[None], emb[None]
